# baseline (speedup 1.0000x reference)
import functools
import numpy as np
import jax
import jax.numpy as jnp
from jax import lax
from jax.experimental import pallas as pl
from jax.experimental.pallas import tpu as pltpu
from jax.experimental.pallas import tpu_sc as plsc

B = 128
NB = 202
NCH = 200
P = 25
NPIX = P * P
PADN = 640
NCHUNK = PADN // 16
CB = 40
NBLK = NCH // CB
NWORK = 32
BPW = B // NWORK

_rng = np.random.RandomState(0)
_size = int(NPIX * 0.08)
_idx_all = np.stack(
    [_rng.choice(np.arange(NPIX), replace=False, size=_size) for _ in range(B)],
    axis=0)
_forced_np = np.zeros((B, PADN), dtype=np.int32)
_forced_np[np.arange(B)[:, None], _idx_all] = 1
_FORCED = _forced_np

_PY = ((np.arange(PADN, dtype=np.int32) // P) % P).copy()
_PX = (np.arange(PADN, dtype=np.int32) % P).copy()

_maskx_np = np.zeros((3, PADN), dtype=np.float32)
_masky_np = np.zeros((3, PADN), dtype=np.float32)
for _d in range(3):
    for _p in range(PADN):
        _y, _x = _p // P, _p % P
        if 0 <= (_p % P) + _d - 1 < P:
            _maskx_np[_d, _p] = 1.0
        if _p < NPIX and 0 <= (_p // P) + _d - 1 < P:
            _masky_np[_d, _p] = 1.0
_MASKX = _maskx_np
_MASKY = _masky_np


def _sc_body(x4, forced, py_t, px_t, xo, sp_v, forced_v, py_v, px_v,
             m_v, rz_v, oy_v, ox_v, gy_v, gx_v,
             in_v0, in_v1, out_v0, out_v1, in_sem, out_sem):
    wid = lax.axis_index("s") * 2 + lax.axis_index("c")
    lane = lax.iota(jnp.int32, 16)
    zero16 = jnp.zeros((16,), jnp.int32)
    pltpu.sync_copy(py_t, py_v)
    pltpu.sync_copy(px_t, px_v)

    def batch_body(bi, _):
        b = wid * BPW + bi
        pltpu.sync_copy(x4.at[b, pl.ds(NCH, 1)], sp_v)
        pltpu.sync_copy(forced.at[b], forced_v)
        cvec = plsc.load_gather(sp_v, [zero16, zero16 + 12, zero16 + 12])
        c_is0 = cvec == 0.0
        last = zero16 + 15

        def chunk_a(j, carry):
            c1, c0 = carry
            base = j * 16
            pv = lane + base
            pyj = py_v[pl.ds(base, 16)]
            pxj = px_v[pl.ds(base, 16)]
            spj = plsc.load_gather(sp_v, [zero16, pyj, pxj])
            fj = forced_v[pl.ds(base, 16)]
            inb = pv < NPIX
            mbool = ((spj == cvec) | c_is0 | (fj > 0)) & inb
            mj = mbool.astype(jnp.int32)
            zj = (inb & (~mbool)).astype(jnp.int32)
            cum1 = plsc.cumsum(mj) + c1
            cum0 = plsc.cumsum(zj) + c0
            m_v[pl.ds(base, 16)] = mj
            rz_v[pl.ds(base, 16)] = cum0 - zj
            ro = cum1 - mj
            plsc.store_scatter(oy_v, [ro], pyj, mask=mbool)
            plsc.store_scatter(ox_v, [ro], pxj, mask=mbool)
            return (jnp.take_along_axis(cum1, last, axis=0),
                    jnp.take_along_axis(cum0, last, axis=0))

        n1vec, _unused = lax.fori_loop(0, NCHUNK, chunk_a, (zero16, zero16))

        @plsc.parallel_loop(0, NCHUNK, unroll=8)
        def chunk_b(j):
            base = j * 16
            mj = m_v[pl.ds(base, 16)] > 0
            t = lax.rem(rz_v[pl.ds(base, 16)], n1vec)
            gy_v[pl.ds(base, 16)] = jnp.where(
                mj, py_v[pl.ds(base, 16)], plsc.load_gather(oy_v, [t]))
            gx_v[pl.ds(base, 16)] = jnp.where(
                mj, px_v[pl.ds(base, 16)], plsc.load_gather(ox_v, [t]))

        in_bufs = (in_v0, in_v1)
        out_bufs = (out_v0, out_v1)

        def start_in(cb, buf):
            return pltpu.async_copy(x4.at[b, pl.ds(cb * CB, CB)], buf, in_sem)

        in_h = {0: start_in(0, in_bufs[0])}
        out_h = {}
        for cb in range(NBLK):
            ib = in_bufs[cb % 2]
            ob = out_bufs[cb % 2]
            in_h[cb % 2].wait()
            if cb + 1 < NBLK:
                in_h[(cb + 1) % 2] = start_in(cb + 1, in_bufs[(cb + 1) % 2])
            if cb % 2 in out_h:
                out_h[cb % 2].wait()

            @plsc.parallel_loop(0, CB * NCHUNK, unroll=8)
            def gather_t(t, ib=ib, ob=ob):
                k = t // NCHUNK
                base = (t - k * NCHUNK) * 16
                gyj = gy_v[pl.ds(base, 16)]
                gxj = gx_v[pl.ds(base, 16)]
                vals = plsc.load_gather(ib, [zero16 + k, gyj, gxj])
                ob[pl.ds(k * PADN + base, 16)] = vals

            out_h[cb % 2] = pltpu.async_copy(
                ob, xo.at[b, pl.ds(cb * CB * PADN, CB * PADN)], out_sem)
        out_h[(NBLK - 1) % 2].wait()
        out_h[(NBLK - 2) % 2].wait()
        return 0

    lax.fori_loop(0, BPW, batch_body, 0)


@functools.cache
def _sc_gather():
    mesh = plsc.VectorSubcoreMesh(core_axis_name="c", subcore_axis_name="s")
    return pl.kernel(
        _sc_body,
        mesh=mesh,
        compiler_params=pltpu.CompilerParams(
            use_tc_tiling_on_sc=False, needs_layout_passes=False),
        out_type=jax.ShapeDtypeStruct((B, NCH * PADN), jnp.float32),
        scratch_types=[
            pltpu.VMEM((1, P, P), jnp.float32),
            pltpu.VMEM((PADN,), jnp.int32),
            pltpu.VMEM((PADN,), jnp.int32),
            pltpu.VMEM((PADN,), jnp.int32),
            pltpu.VMEM((PADN,), jnp.int32),
            pltpu.VMEM((PADN,), jnp.int32),
            pltpu.VMEM((PADN,), jnp.int32),
            pltpu.VMEM((PADN,), jnp.int32),
            pltpu.VMEM((PADN,), jnp.int32),
            pltpu.VMEM((PADN,), jnp.int32),
            pltpu.VMEM((CB, P, P), jnp.float32),
            pltpu.VMEM((CB, P, P), jnp.float32),
            pltpu.VMEM((CB * PADN,), jnp.float32),
            pltpu.VMEM((CB * PADN,), jnp.float32),
            pltpu.SemaphoreType.DMA,
            pltpu.SemaphoreType.DMA,
        ],
    )


def _rot(a, s):
    if s > 0:
        return jnp.concatenate([a[:, s:], a[:, :s]], axis=1)
    if s < 0:
        return jnp.concatenate([a[:, s:], a[:, :PADN + s]], axis=1)
    return a


def _conv_body(xo_ref, wt_ref, maskx_ref, masky_ref, b_ref, out_ref):
    x2 = xo_ref[0]
    us = [_rot(x2, dx) * maskx_ref[dx + 1][None, :] for dx in (-1, 0, 1)]
    acc = b_ref[...].astype(jnp.float32) * jnp.ones((32, PADN), jnp.float32)
    for dy in (-1, 0, 1):
        part = jnp.zeros((32, PADN), jnp.float32)
        for dx in (-1, 0, 1):
            k = (dy + 1) * 3 + (dx + 1)
            part = part + lax.dot_general(
                wt_ref[k], us[dx + 1], (((1,), (0,)), ((), ())),
                preferred_element_type=jnp.float32)
        acc = acc + _rot(part, dy * P) * masky_ref[dy + 1][None, :]
    out_ref[0] = acc[:, :NPIX]


_conv = pl.pallas_call(
    _conv_body,
    grid=(B,),
    in_specs=[
        pl.BlockSpec((1, NCH, PADN), lambda i: (i, 0, 0)),
        pl.BlockSpec((9, 32, NCH), lambda i: (0, 0, 0)),
        pl.BlockSpec((3, PADN), lambda i: (0, 0)),
        pl.BlockSpec((3, PADN), lambda i: (0, 0)),
        pl.BlockSpec((32, 1), lambda i: (0, 0)),
    ],
    out_specs=pl.BlockSpec((1, 32, NPIX), lambda i: (i, 0, 0)),
    out_shape=jax.ShapeDtypeStruct((B, 32, NPIX), jnp.float32),
)


def kernel(x, W, b):
    xo = _sc_gather()(x, jnp.asarray(_FORCED), jnp.asarray(_PY),
                      jnp.asarray(_PX))
    xo3 = jnp.reshape(xo, (B, NCH, PADN))
    wt = jnp.reshape(jnp.transpose(W, (2, 3, 0, 1)), (9, 32, NCH))
    y = _conv(xo3, wt, jnp.asarray(_MASKX), jnp.asarray(_MASKY),
              jnp.reshape(b, (32, 1)))
    return jnp.reshape(y, (B, 32, P, P))


# speedup vs baseline: 3.0906x; 3.0906x over previous
import functools
import numpy as np
import jax
import jax.numpy as jnp
from jax import lax
from jax.experimental import pallas as pl
from jax.experimental.pallas import tpu as pltpu
from jax.experimental.pallas import tpu_sc as plsc

B = 128
NB = 202
NCH = 200
P = 25
NPIX = P * P
PADN = 640
NCHUNK = PADN // 16
CB = 40
NBLK = NCH // CB
NWORK = 32
BPW = B // NWORK

_rng = np.random.RandomState(0)
_size = int(NPIX * 0.08)
_idx_all = np.stack(
    [_rng.choice(np.arange(NPIX), replace=False, size=_size) for _ in range(B)],
    axis=0)
_forced_np = np.zeros((B, 1, PADN), dtype=np.int32)
_forced_np[np.arange(B)[:, None], 0, _idx_all] = 1
_FORCED = _forced_np

_maskx_np = np.zeros((3, PADN), dtype=np.float32)
_masky_np = np.zeros((3, PADN), dtype=np.float32)
for _d in range(3):
    for _p in range(PADN):
        _y, _x = _p // P, _p % P
        if 0 <= (_p % P) + _d - 1 < P:
            _maskx_np[_d, _p] = 1.0
        if _p < NPIX and 0 <= (_p // P) + _d - 1 < P:
            _masky_np[_d, _p] = 1.0
_MASKX = _maskx_np
_MASKY = _masky_np


def _sc_body(x_rows, forced, xo, sp_v, forced_v, m_v, rz_v, onepos_v, g_v,
             in_v0, in_v1, out_v0, out_v1, in_sem, out_sem):
    wid = lax.axis_index("s") * 2 + lax.axis_index("c")
    lane = lax.iota(jnp.int32, 16)

    def batch_body(bi, _):
        b = wid * BPW + bi
        pltpu.sync_copy(x_rows.at[b, pl.ds(NCH, 1)], sp_v)
        pltpu.sync_copy(forced.at[b], forced_v)
        zero16 = jnp.zeros((16,), jnp.int32)
        cvec = plsc.load_gather(sp_v, [zero16, zero16 + 312])
        c_is0 = cvec == 0.0
        last = jnp.zeros((16,), jnp.int32) + 15

        def chunk_a(j, carry):
            c1, c0 = carry
            base = j * 16
            pv = lane + base
            spj = sp_v[0, pl.ds(base, 16)]
            fj = forced_v[0, pl.ds(base, 16)]
            inb = pv < NPIX
            mbool = ((spj == cvec) | c_is0 | (fj > 0)) & inb
            mj = mbool.astype(jnp.int32)
            zj = (inb & (~mbool)).astype(jnp.int32)
            cum1 = plsc.cumsum(mj) + c1
            cum0 = plsc.cumsum(zj) + c0
            m_v[pl.ds(base, 16)] = mj
            rz_v[pl.ds(base, 16)] = cum0 - zj
            plsc.store_scatter(onepos_v, [cum1 - mj], pv, mask=mbool)
            return (jnp.take_along_axis(cum1, last, axis=0),
                    jnp.take_along_axis(cum0, last, axis=0))

        n1vec, _unused = lax.fori_loop(
            0, NCHUNK, chunk_a,
            (jnp.zeros((16,), jnp.int32), jnp.zeros((16,), jnp.int32)))

        @plsc.parallel_loop(0, NCHUNK, unroll=8)
        def chunk_b(j):
            base = j * 16
            pv = lane + base
            mj = m_v[pl.ds(base, 16)]
            t = lax.rem(rz_v[pl.ds(base, 16)], n1vec)
            src = plsc.load_gather(onepos_v, [t])
            g_v[pl.ds(base, 16)] = jnp.where(mj > 0, pv, src)

        in_bufs = (in_v0, in_v1)
        out_bufs = (out_v0, out_v1)

        def start_in(cb, buf):
            return pltpu.async_copy(
                x_rows.at[b, pl.ds(cb * CB, CB)], buf, in_sem)

        in_h = {0: start_in(0, in_bufs[0])}
        out_h = {}
        for cb in range(NBLK):
            ib = in_bufs[cb % 2]
            ob = out_bufs[cb % 2]
            in_h[cb % 2].wait()
            if cb + 1 < NBLK:
                in_h[(cb + 1) % 2] = start_in(cb + 1, in_bufs[(cb + 1) % 2])
            if cb % 2 in out_h:
                out_h[cb % 2].wait()

            @plsc.parallel_loop(0, CB * NCHUNK, unroll=8)
            def gather_t(t, ib=ib, ob=ob):
                k = t // NCHUNK
                base = (t - k * NCHUNK) * 16
                gj = g_v[pl.ds(base, 16)]
                vals = plsc.load_gather(
                    ib, [jnp.zeros((16,), jnp.int32) + k, gj])
                ob[k, pl.ds(base, 16)] = vals

            out_h[cb % 2] = pltpu.async_copy(
                ob, xo.at[b, pl.ds(cb * CB, CB)], out_sem)
        out_h[(NBLK - 1) % 2].wait()
        out_h[(NBLK - 2) % 2].wait()
        return 0

    lax.fori_loop(0, BPW, batch_body, 0)


@functools.cache
def _sc_gather():
    mesh = plsc.VectorSubcoreMesh(core_axis_name="c", subcore_axis_name="s")
    return pl.kernel(
        _sc_body,
        mesh=mesh,
        compiler_params=pltpu.CompilerParams(needs_layout_passes=False),
        out_type=jax.ShapeDtypeStruct((B, NCH, PADN), jnp.float32),
        scratch_types=[
            pltpu.VMEM((1, NPIX), jnp.float32),
            pltpu.VMEM((1, PADN), jnp.int32),
            pltpu.VMEM((PADN,), jnp.int32),
            pltpu.VMEM((PADN,), jnp.int32),
            pltpu.VMEM((PADN,), jnp.int32),
            pltpu.VMEM((PADN,), jnp.int32),
            pltpu.VMEM((CB, NPIX), jnp.float32),
            pltpu.VMEM((CB, NPIX), jnp.float32),
            pltpu.VMEM((CB, PADN), jnp.float32),
            pltpu.VMEM((CB, PADN), jnp.float32),
            pltpu.SemaphoreType.DMA,
            pltpu.SemaphoreType.DMA,
        ],
    )


def _rot(a, s):
    if s > 0:
        return jnp.concatenate([a[:, s:], a[:, :s]], axis=1)
    if s < 0:
        return jnp.concatenate([a[:, s:], a[:, :PADN + s]], axis=1)
    return a


def _conv_body(xo_ref, wt_ref, maskx_ref, masky_ref, b_ref, out_ref):
    x2 = xo_ref[0]
    us = [_rot(x2, dx) * maskx_ref[dx + 1][None, :] for dx in (-1, 0, 1)]
    acc = b_ref[...].astype(jnp.float32) * jnp.ones((32, PADN), jnp.float32)
    for dy in (-1, 0, 1):
        part = jnp.zeros((32, PADN), jnp.float32)
        for dx in (-1, 0, 1):
            k = (dy + 1) * 3 + (dx + 1)
            part = part + lax.dot_general(
                wt_ref[k], us[dx + 1], (((1,), (0,)), ((), ())),
                preferred_element_type=jnp.float32)
        acc = acc + _rot(part, dy * P) * masky_ref[dy + 1][None, :]
    out_ref[0] = acc[:, :NPIX]


_conv = pl.pallas_call(
    _conv_body,
    grid=(B,),
    in_specs=[
        pl.BlockSpec((1, NCH, PADN), lambda i: (i, 0, 0)),
        pl.BlockSpec((9, 32, NCH), lambda i: (0, 0, 0)),
        pl.BlockSpec((3, PADN), lambda i: (0, 0)),
        pl.BlockSpec((3, PADN), lambda i: (0, 0)),
        pl.BlockSpec((32, 1), lambda i: (0, 0)),
    ],
    out_specs=pl.BlockSpec((1, 32, NPIX), lambda i: (i, 0, 0)),
    out_shape=jax.ShapeDtypeStruct((B, 32, NPIX), jnp.float32),
)


def kernel(x, W, b):
    x3 = jnp.reshape(x, (B, NB, NPIX))
    xo3 = _sc_gather()(x3, jnp.asarray(_FORCED))
    wt = jnp.reshape(jnp.transpose(W, (2, 3, 0, 1)), (9, 32, NCH))
    y = _conv(xo3, wt, jnp.asarray(_MASKX), jnp.asarray(_MASKY),
              jnp.reshape(b, (32, 1)))
    return jnp.reshape(y, (B, 32, P, P))


# speedup vs baseline: 3.3145x; 1.0725x over previous
import functools
import numpy as np
import jax
import jax.numpy as jnp
from jax import lax
from jax.experimental import pallas as pl
from jax.experimental.pallas import tpu as pltpu
from jax.experimental.pallas import tpu_sc as plsc

B = 128
NB = 202
NCH = 200
P = 25
NPIX = P * P
PADN = 640
NCHUNK = PADN // 16
CB = 40
NBLK = NCH // CB
NWORK = 32
BPW = B // NWORK

_rng = np.random.RandomState(0)
_size = int(NPIX * 0.08)
_idx_all = np.stack(
    [_rng.choice(np.arange(NPIX), replace=False, size=_size) for _ in range(B)],
    axis=0)
_forced_np = np.zeros((B, 1, PADN), dtype=np.int32)
_forced_np[np.arange(B)[:, None], 0, _idx_all] = 1
_FORCED = _forced_np

_maskx_np = np.zeros((3, PADN), dtype=np.float32)
_masky_np = np.zeros((3, PADN), dtype=np.float32)
for _d in range(3):
    for _p in range(PADN):
        _y, _x = _p // P, _p % P
        if 0 <= (_p % P) + _d - 1 < P:
            _maskx_np[_d, _p] = 1.0
        if _p < NPIX and 0 <= (_p // P) + _d - 1 < P:
            _masky_np[_d, _p] = 1.0
_MASKX = _maskx_np
_MASKY = _masky_np


def _sc_body(b0, nb_half, x_rows, forced, xo, sp_v, forced_v, m_v, rz_v,
             onepos_v, g_v, in_v0, in_v1, out_v0, out_v1, in_sem, out_sem):
    bpw = nb_half // NWORK
    wid = lax.axis_index("s") * 2 + lax.axis_index("c")
    lane = lax.iota(jnp.int32, 16)

    def batch_body(bi, _):
        bl = wid * bpw + bi
        b = b0 + bl
        pltpu.sync_copy(x_rows.at[b, pl.ds(NCH, 1)], sp_v)
        pltpu.sync_copy(forced.at[b], forced_v)
        zero16 = jnp.zeros((16,), jnp.int32)
        cvec = plsc.load_gather(sp_v, [zero16, zero16 + 312])
        c_is0 = cvec == 0.0
        last = jnp.zeros((16,), jnp.int32) + 15

        def chunk_a(j, carry):
            c1, c0 = carry
            base = j * 16
            pv = lane + base
            spj = sp_v[0, pl.ds(base, 16)]
            fj = forced_v[0, pl.ds(base, 16)]
            inb = pv < NPIX
            mbool = ((spj == cvec) | c_is0 | (fj > 0)) & inb
            mj = mbool.astype(jnp.int32)
            zj = (inb & (~mbool)).astype(jnp.int32)
            cum1 = plsc.cumsum(mj) + c1
            cum0 = plsc.cumsum(zj) + c0
            m_v[pl.ds(base, 16)] = mj
            rz_v[pl.ds(base, 16)] = cum0 - zj
            plsc.store_scatter(onepos_v, [cum1 - mj], pv, mask=mbool)
            return (jnp.take_along_axis(cum1, last, axis=0),
                    jnp.take_along_axis(cum0, last, axis=0))

        n1vec, _unused = lax.fori_loop(
            0, NCHUNK, chunk_a,
            (jnp.zeros((16,), jnp.int32), jnp.zeros((16,), jnp.int32)))

        @plsc.parallel_loop(0, NCHUNK, unroll=8)
        def chunk_b(j):
            base = j * 16
            pv = lane + base
            mj = m_v[pl.ds(base, 16)]
            t = lax.rem(rz_v[pl.ds(base, 16)], n1vec)
            src = plsc.load_gather(onepos_v, [t])
            g_v[pl.ds(base, 16)] = jnp.where(mj > 0, pv, src)

        in_bufs = (in_v0, in_v1)
        out_bufs = (out_v0, out_v1)

        def start_in(cb, buf):
            return pltpu.async_copy(
                x_rows.at[b, pl.ds(cb * CB, CB)], buf, in_sem)

        in_h = {0: start_in(0, in_bufs[0])}
        out_h = {}
        for cb in range(NBLK):
            ib = in_bufs[cb % 2]
            ob = out_bufs[cb % 2]
            in_h[cb % 2].wait()
            if cb + 1 < NBLK:
                in_h[(cb + 1) % 2] = start_in(cb + 1, in_bufs[(cb + 1) % 2])
            if cb % 2 in out_h:
                out_h[cb % 2].wait()

            @plsc.parallel_loop(0, CB * NCHUNK, unroll=8)
            def gather_t(t, ib=ib, ob=ob):
                k = t // NCHUNK
                base = (t - k * NCHUNK) * 16
                gj = g_v[pl.ds(base, 16)]
                vals = plsc.load_gather(
                    ib, [jnp.zeros((16,), jnp.int32) + k, gj])
                ob[k, pl.ds(base, 16)] = vals

            out_h[cb % 2] = pltpu.async_copy(
                ob, xo.at[bl, pl.ds(cb * CB, CB)], out_sem)
        out_h[(NBLK - 1) % 2].wait()
        out_h[(NBLK - 2) % 2].wait()
        return 0

    lax.fori_loop(0, bpw, batch_body, 0)


@functools.cache
def _sc_gather(b0, nb_half):
    mesh = plsc.VectorSubcoreMesh(core_axis_name="c", subcore_axis_name="s")
    return pl.kernel(
        functools.partial(_sc_body, b0, nb_half),
        mesh=mesh,
        compiler_params=pltpu.CompilerParams(needs_layout_passes=False),
        out_type=jax.ShapeDtypeStruct((nb_half, NCH, PADN), jnp.float32),
        scratch_types=[
            pltpu.VMEM((1, NPIX), jnp.float32),
            pltpu.VMEM((1, PADN), jnp.int32),
            pltpu.VMEM((PADN,), jnp.int32),
            pltpu.VMEM((PADN,), jnp.int32),
            pltpu.VMEM((PADN,), jnp.int32),
            pltpu.VMEM((PADN,), jnp.int32),
            pltpu.VMEM((CB, NPIX), jnp.float32),
            pltpu.VMEM((CB, NPIX), jnp.float32),
            pltpu.VMEM((CB, PADN), jnp.float32),
            pltpu.VMEM((CB, PADN), jnp.float32),
            pltpu.SemaphoreType.DMA,
            pltpu.SemaphoreType.DMA,
        ],
    )


def _rot(a, s):
    if s > 0:
        return jnp.concatenate([a[:, s:], a[:, :s]], axis=1)
    if s < 0:
        return jnp.concatenate([a[:, s:], a[:, :PADN + s]], axis=1)
    return a


def _conv_body(xo_ref, wt_ref, maskx_ref, masky_ref, b_ref, out_ref):
    x2 = xo_ref[0]
    us = [_rot(x2, dx) * maskx_ref[dx + 1][None, :] for dx in (-1, 0, 1)]
    acc = b_ref[...].astype(jnp.float32) * jnp.ones((32, PADN), jnp.float32)
    for dy in (-1, 0, 1):
        part = jnp.zeros((32, PADN), jnp.float32)
        for dx in (-1, 0, 1):
            k = (dy + 1) * 3 + (dx + 1)
            part = part + lax.dot_general(
                wt_ref[k], us[dx + 1], (((1,), (0,)), ((), ())),
                preferred_element_type=jnp.float32)
        acc = acc + _rot(part, dy * P) * masky_ref[dy + 1][None, :]
    out_ref[0] = acc[:, :NPIX]


@functools.cache
def _conv(nb):
    return pl.pallas_call(
    _conv_body,
    grid=(nb,),
    in_specs=[
        pl.BlockSpec((1, NCH, PADN), lambda i: (i, 0, 0)),
        pl.BlockSpec((9, 32, NCH), lambda i: (0, 0, 0)),
        pl.BlockSpec((3, PADN), lambda i: (0, 0)),
        pl.BlockSpec((3, PADN), lambda i: (0, 0)),
        pl.BlockSpec((32, 1), lambda i: (0, 0)),
    ],
    out_specs=pl.BlockSpec((1, 32, NPIX), lambda i: (i, 0, 0)),
    out_shape=jax.ShapeDtypeStruct((nb, 32, NPIX), jnp.float32),
    )


def kernel(x, W, b):
    x3 = jnp.reshape(x, (B, NB, NPIX))
    wt = jnp.reshape(jnp.transpose(W, (2, 3, 0, 1)), (9, 32, NCH))
    mx, my = jnp.asarray(_MASKX), jnp.asarray(_MASKY)
    b2 = jnp.reshape(b, (32, 1))
    forced = jnp.asarray(_FORCED)
    half = B // 2
    xo_a = _sc_gather(0, half)(x3, forced)
    xo_b = _sc_gather(half, half)(x3, forced)
    y_a = _conv(half)(xo_a, wt, mx, my, b2)
    y_b = _conv(half)(xo_b, wt, mx, my, b2)
    y = jnp.concatenate([y_a, y_b], axis=0)
    return jnp.reshape(y, (B, 32, P, P))


# speedup vs baseline: 3.9553x; 1.1933x over previous
import functools
import numpy as np
import jax
import jax.numpy as jnp
from jax import lax
from jax.experimental import pallas as pl
from jax.experimental.pallas import tpu as pltpu
from jax.experimental.pallas import tpu_sc as plsc

B = 128
NB = 202
NCH = 200
P = 25
NPIX = P * P
PADN = 640
NCHUNK = PADN // 16
CB = 40
NBLK = NCH // CB
NWORK = 32
BPW = B // NWORK

_rng = np.random.RandomState(0)
_size = int(NPIX * 0.08)
_idx_all = np.stack(
    [_rng.choice(np.arange(NPIX), replace=False, size=_size) for _ in range(B)],
    axis=0)
_forced_np = np.zeros((B, 1, PADN), dtype=np.int32)
_forced_np[np.arange(B)[:, None], 0, _idx_all] = 1
_FORCED = _forced_np

_maskx_np = np.zeros((3, PADN), dtype=np.float32)
_masky_np = np.zeros((3, PADN), dtype=np.float32)
for _d in range(3):
    for _p in range(PADN):
        _y, _x = _p // P, _p % P
        if 0 <= (_p % P) + _d - 1 < P:
            _maskx_np[_d, _p] = 1.0
        if _p < NPIX and 0 <= (_p // P) + _d - 1 < P:
            _masky_np[_d, _p] = 1.0
_MASKX = _maskx_np
_MASKY = _masky_np


def _sc_body(b0, nb_half, x_rows, forced, xo, sp_v, forced_v, m_v, rz_v,
             onepos_v, g_v, in_v0, in_v1, out_v0, out_v1, in_sem, out_sem):
    bpw = nb_half // NWORK
    wid = lax.axis_index("s") * 2 + lax.axis_index("c")
    lane = lax.iota(jnp.int32, 16)

    def batch_body(bi, _):
        bl = wid * bpw + bi
        b = b0 + bl
        pltpu.sync_copy(x_rows.at[b, pl.ds(NCH, 1)], sp_v)
        pltpu.sync_copy(forced.at[b], forced_v)
        zero16 = jnp.zeros((16,), jnp.int32)
        cvec = plsc.load_gather(sp_v, [zero16, zero16 + 312])
        c_is0 = cvec == 0.0
        last = jnp.zeros((16,), jnp.int32) + 15

        def chunk_a(j, carry):
            c1, c0 = carry
            base = j * 16
            pv = lane + base
            spj = sp_v[0, pl.ds(base, 16)]
            fj = forced_v[0, pl.ds(base, 16)]
            inb = pv < NPIX
            mbool = ((spj == cvec) | c_is0 | (fj > 0)) & inb
            mj = mbool.astype(jnp.int32)
            zj = (inb & (~mbool)).astype(jnp.int32)
            cum1 = plsc.cumsum(mj) + c1
            cum0 = plsc.cumsum(zj) + c0
            m_v[pl.ds(base, 16)] = mj
            rz_v[pl.ds(base, 16)] = cum0 - zj
            plsc.store_scatter(onepos_v, [cum1 - mj], pv, mask=mbool)
            return (jnp.take_along_axis(cum1, last, axis=0),
                    jnp.take_along_axis(cum0, last, axis=0))

        n1vec, _unused = lax.fori_loop(
            0, NCHUNK, chunk_a,
            (jnp.zeros((16,), jnp.int32), jnp.zeros((16,), jnp.int32)))

        @plsc.parallel_loop(0, NCHUNK, unroll=8)
        def chunk_b(j):
            base = j * 16
            pv = lane + base
            mj = m_v[pl.ds(base, 16)]
            t = lax.rem(rz_v[pl.ds(base, 16)], n1vec)
            src = plsc.load_gather(onepos_v, [t])
            g_v[pl.ds(base, 16)] = jnp.where(mj > 0, pv, src)

        in_bufs = (in_v0, in_v1)
        out_bufs = (out_v0, out_v1)

        def start_in(cb, buf):
            return pltpu.async_copy(
                x_rows.at[b, pl.ds(cb * CB, CB)], buf, in_sem)

        in_h = {0: start_in(0, in_bufs[0])}
        out_h = {}
        for cb in range(NBLK):
            ib = in_bufs[cb % 2]
            ob = out_bufs[cb % 2]
            in_h[cb % 2].wait()
            if cb + 1 < NBLK:
                in_h[(cb + 1) % 2] = start_in(cb + 1, in_bufs[(cb + 1) % 2])
            if cb % 2 in out_h:
                out_h[cb % 2].wait()

            @plsc.parallel_loop(0, CB * NCHUNK, unroll=8)
            def gather_t(t, ib=ib, ob=ob):
                k = t // NCHUNK
                base = (t - k * NCHUNK) * 16
                gj = g_v[pl.ds(base, 16)]
                vals = plsc.load_gather(
                    ib, [jnp.zeros((16,), jnp.int32) + k, gj])
                ob[k, pl.ds(base, 16)] = vals

            out_h[cb % 2] = pltpu.async_copy(
                ob, xo.at[bl, pl.ds(cb * CB, CB)], out_sem)
        out_h[(NBLK - 1) % 2].wait()
        out_h[(NBLK - 2) % 2].wait()
        return 0

    lax.fori_loop(0, bpw, batch_body, 0)


@functools.cache
def _sc_gather(b0, nb_half):
    mesh = plsc.VectorSubcoreMesh(core_axis_name="c", subcore_axis_name="s")
    return pl.kernel(
        functools.partial(_sc_body, b0, nb_half),
        mesh=mesh,
        compiler_params=pltpu.CompilerParams(needs_layout_passes=False),
        out_type=jax.ShapeDtypeStruct((nb_half, NCH, PADN), jnp.float32),
        scratch_types=[
            pltpu.VMEM((1, NPIX), jnp.float32),
            pltpu.VMEM((1, PADN), jnp.int32),
            pltpu.VMEM((PADN,), jnp.int32),
            pltpu.VMEM((PADN,), jnp.int32),
            pltpu.VMEM((PADN,), jnp.int32),
            pltpu.VMEM((PADN,), jnp.int32),
            pltpu.VMEM((CB, NPIX), jnp.float32),
            pltpu.VMEM((CB, NPIX), jnp.float32),
            pltpu.VMEM((CB, PADN), jnp.float32),
            pltpu.VMEM((CB, PADN), jnp.float32),
            pltpu.SemaphoreType.DMA,
            pltpu.SemaphoreType.DMA,
        ],
    )


def _rot(a, s):
    if s > 0:
        return jnp.concatenate([a[:, s:], a[:, :s]], axis=1)
    if s < 0:
        return jnp.concatenate([a[:, s:], a[:, :PADN + s]], axis=1)
    return a


CONV_BB = 2


def _conv_body(xo_ref, wt_ref, maskx_ref, masky_ref, b_ref, out_ref):
    for bb in range(CONV_BB):
        x2 = xo_ref[bb]
        us = [_rot(x2, dx) * maskx_ref[dx + 1][None, :] for dx in (-1, 0, 1)]
        acc = b_ref[...].astype(jnp.float32) * jnp.ones(
            (32, PADN), jnp.float32)
        for dy in (-1, 0, 1):
            part = jnp.zeros((32, PADN), jnp.float32)
            for dx in (-1, 0, 1):
                k = (dy + 1) * 3 + (dx + 1)
                part = part + lax.dot_general(
                    wt_ref[k], us[dx + 1], (((1,), (0,)), ((), ())),
                    preferred_element_type=jnp.float32)
            acc = acc + _rot(part, dy * P) * masky_ref[dy + 1][None, :]
        out_ref[bb] = acc[:, :NPIX]


@functools.cache
def _conv(nb):
    return pl.pallas_call(
    _conv_body,
    grid=(nb // CONV_BB,),
    in_specs=[
        pl.BlockSpec((CONV_BB, NCH, PADN), lambda i: (i, 0, 0)),
        pl.BlockSpec((9, 32, NCH), lambda i: (0, 0, 0)),
        pl.BlockSpec((3, PADN), lambda i: (0, 0)),
        pl.BlockSpec((3, PADN), lambda i: (0, 0)),
        pl.BlockSpec((32, 1), lambda i: (0, 0)),
    ],
    out_specs=pl.BlockSpec((CONV_BB, 32, NPIX), lambda i: (i, 0, 0)),
    out_shape=jax.ShapeDtypeStruct((nb, 32, NPIX), jnp.float32),
    )


def kernel(x, W, b):
    x3 = jnp.reshape(x, (B, NB, NPIX))
    wt = jnp.reshape(jnp.transpose(W, (2, 3, 0, 1)), (9, 32, NCH))
    mx, my = jnp.asarray(_MASKX), jnp.asarray(_MASKY)
    b2 = jnp.reshape(b, (32, 1))
    forced = jnp.asarray(_FORCED)
    half = B // 2
    xo_a = _sc_gather(0, half)(x3, forced)
    xo_b = _sc_gather(half, half)(x3, forced)
    y_a = _conv(half)(xo_a, wt, mx, my, b2)
    y_b = _conv(half)(xo_b, wt, mx, my, b2)
    y = jnp.concatenate([y_a, y_b], axis=0)
    return jnp.reshape(y, (B, 32, P, P))


# speedup vs baseline: 4.1247x; 1.0428x over previous
import functools
import numpy as np
import jax
import jax.numpy as jnp
from jax import lax
from jax.experimental import pallas as pl
from jax.experimental.pallas import tpu as pltpu
from jax.experimental.pallas import tpu_sc as plsc

B = 128
NB = 202
NCH = 200
P = 25
NPIX = P * P
PADN = 640
NCHUNK = PADN // 16
CB = 40
NBLK = NCH // CB
NWORK = 32
BPW = B // NWORK

_rng = np.random.RandomState(0)
_size = int(NPIX * 0.08)
_idx_all = np.stack(
    [_rng.choice(np.arange(NPIX), replace=False, size=_size) for _ in range(B)],
    axis=0)
_forced_np = np.zeros((B, 1, PADN), dtype=np.int32)
_forced_np[np.arange(B)[:, None], 0, _idx_all] = 1
_FORCED = _forced_np

_maskx_np = np.zeros((3, PADN), dtype=np.float32)
_masky_np = np.zeros((3, PADN), dtype=np.float32)
for _d in range(3):
    for _p in range(PADN):
        _y, _x = _p // P, _p % P
        if 0 <= (_p % P) + _d - 1 < P:
            _maskx_np[_d, _p] = 1.0
        if _p < NPIX and 0 <= (_p // P) + _d - 1 < P:
            _masky_np[_d, _p] = 1.0
_MASKX = _maskx_np
_MASKY = _masky_np


def _sc_body(b0, nb_half, x_rows, forced, xo, sp_v, forced_v, m_v, rz_v,
             onepos_v, g_v, in_v0, in_v1, out_v0, out_v1, in_sem, out_sem):
    bpw = nb_half // NWORK
    wid = lax.axis_index("s") * 2 + lax.axis_index("c")
    lane = lax.iota(jnp.int32, 16)

    def batch_body(bi, _):
        bl = wid * bpw + bi
        b = b0 + bl
        pltpu.sync_copy(x_rows.at[b, pl.ds(NCH, 1)], sp_v)
        pltpu.sync_copy(forced.at[b], forced_v)
        zero16 = jnp.zeros((16,), jnp.int32)
        cvec = plsc.load_gather(sp_v, [zero16, zero16 + 312])
        c_is0 = cvec == 0.0
        last = jnp.zeros((16,), jnp.int32) + 15

        def chunk_a(j, carry):
            c1, c0 = carry
            base = j * 16
            pv = lane + base
            spj = sp_v[0, pl.ds(base, 16)]
            fj = forced_v[0, pl.ds(base, 16)]
            inb = pv < NPIX
            mbool = ((spj == cvec) | c_is0 | (fj > 0)) & inb
            mj = mbool.astype(jnp.int32)
            zj = (inb & (~mbool)).astype(jnp.int32)
            cum1 = plsc.cumsum(mj) + c1
            cum0 = plsc.cumsum(zj) + c0
            m_v[pl.ds(base, 16)] = mj
            rz_v[pl.ds(base, 16)] = cum0 - zj
            plsc.store_scatter(onepos_v, [cum1 - mj], pv, mask=mbool)
            return (jnp.take_along_axis(cum1, last, axis=0),
                    jnp.take_along_axis(cum0, last, axis=0))

        n1vec, _unused = lax.fori_loop(
            0, NCHUNK, chunk_a,
            (jnp.zeros((16,), jnp.int32), jnp.zeros((16,), jnp.int32)))

        @plsc.parallel_loop(0, NCHUNK, unroll=8)
        def chunk_b(j):
            base = j * 16
            pv = lane + base
            mj = m_v[pl.ds(base, 16)]
            t = lax.rem(rz_v[pl.ds(base, 16)], n1vec)
            src = plsc.load_gather(onepos_v, [t])
            g_v[pl.ds(base, 16)] = jnp.where(mj > 0, pv, src)

        in_bufs = (in_v0, in_v1)
        out_bufs = (out_v0, out_v1)

        def start_in(cb, buf):
            return pltpu.async_copy(
                x_rows.at[b, pl.ds(cb * CB, CB)], buf, in_sem)

        in_h = {0: start_in(0, in_bufs[0])}
        out_h = {}
        for cb in range(NBLK):
            ib = in_bufs[cb % 2]
            ob = out_bufs[cb % 2]
            in_h[cb % 2].wait()
            if cb + 1 < NBLK:
                in_h[(cb + 1) % 2] = start_in(cb + 1, in_bufs[(cb + 1) % 2])
            if cb % 2 in out_h:
                out_h[cb % 2].wait()

            @plsc.parallel_loop(0, CB * NCHUNK, unroll=8)
            def gather_t(t, ib=ib, ob=ob):
                k = t // NCHUNK
                base = (t - k * NCHUNK) * 16
                gj = g_v[pl.ds(base, 16)]
                vals = plsc.load_gather(
                    ib, [jnp.zeros((16,), jnp.int32) + k, gj])
                ob[k, pl.ds(base, 16)] = vals

            out_h[cb % 2] = pltpu.async_copy(
                ob, xo.at[bl, pl.ds(cb * CB, CB)], out_sem)
        out_h[(NBLK - 1) % 2].wait()
        out_h[(NBLK - 2) % 2].wait()
        return 0

    lax.fori_loop(0, bpw, batch_body, 0)


@functools.cache
def _sc_gather(b0, nb_half):
    mesh = plsc.VectorSubcoreMesh(core_axis_name="c", subcore_axis_name="s")
    return pl.kernel(
        functools.partial(_sc_body, b0, nb_half),
        mesh=mesh,
        compiler_params=pltpu.CompilerParams(needs_layout_passes=False),
        out_type=jax.ShapeDtypeStruct((nb_half, NCH, PADN), jnp.float32),
        scratch_types=[
            pltpu.VMEM((1, NPIX), jnp.float32),
            pltpu.VMEM((1, PADN), jnp.int32),
            pltpu.VMEM((PADN,), jnp.int32),
            pltpu.VMEM((PADN,), jnp.int32),
            pltpu.VMEM((PADN,), jnp.int32),
            pltpu.VMEM((PADN,), jnp.int32),
            pltpu.VMEM((CB, NPIX), jnp.float32),
            pltpu.VMEM((CB, NPIX), jnp.float32),
            pltpu.VMEM((CB, PADN), jnp.float32),
            pltpu.VMEM((CB, PADN), jnp.float32),
            pltpu.SemaphoreType.DMA,
            pltpu.SemaphoreType.DMA,
        ],
    )


def _rot(a, s):
    if s > 0:
        return jnp.concatenate([a[:, s:], a[:, :s]], axis=1)
    if s < 0:
        return jnp.concatenate([a[:, s:], a[:, :PADN + s]], axis=1)
    return a


CONV_BB = 4


def _conv_body(xo_ref, wt_ref, maskx_ref, masky_ref, b_ref, out_ref):
    for bb in range(CONV_BB):
        x2 = xo_ref[bb]
        us = [_rot(x2, dx) * maskx_ref[dx + 1][None, :] for dx in (-1, 0, 1)]
        acc = b_ref[...].astype(jnp.float32) * jnp.ones(
            (32, PADN), jnp.float32)
        for dy in (-1, 0, 1):
            part = jnp.zeros((32, PADN), jnp.float32)
            for dx in (-1, 0, 1):
                k = (dy + 1) * 3 + (dx + 1)
                part = part + lax.dot_general(
                    wt_ref[k], us[dx + 1], (((1,), (0,)), ((), ())),
                    preferred_element_type=jnp.float32)
            acc = acc + _rot(part, dy * P) * masky_ref[dy + 1][None, :]
        out_ref[bb] = acc[:, :NPIX]


@functools.cache
def _conv(nb):
    return pl.pallas_call(
    _conv_body,
    grid=(nb // CONV_BB,),
    in_specs=[
        pl.BlockSpec((CONV_BB, NCH, PADN), lambda i: (i, 0, 0)),
        pl.BlockSpec((9, 32, NCH), lambda i: (0, 0, 0)),
        pl.BlockSpec((3, PADN), lambda i: (0, 0)),
        pl.BlockSpec((3, PADN), lambda i: (0, 0)),
        pl.BlockSpec((32, 1), lambda i: (0, 0)),
    ],
    out_specs=pl.BlockSpec((CONV_BB, 32, NPIX), lambda i: (i, 0, 0)),
    out_shape=jax.ShapeDtypeStruct((nb, 32, NPIX), jnp.float32),
    )


def kernel(x, W, b):
    x3 = jnp.reshape(x, (B, NB, NPIX))
    wt = jnp.reshape(jnp.transpose(W, (2, 3, 0, 1)), (9, 32, NCH))
    mx, my = jnp.asarray(_MASKX), jnp.asarray(_MASKY)
    b2 = jnp.reshape(b, (32, 1))
    forced = jnp.asarray(_FORCED)
    half = B // 2
    xo_a = _sc_gather(0, half)(x3, forced)
    xo_b = _sc_gather(half, half)(x3, forced)
    y_a = _conv(half)(xo_a, wt, mx, my, b2)
    y_b = _conv(half)(xo_b, wt, mx, my, b2)
    y = jnp.concatenate([y_a, y_b], axis=0)
    return jnp.reshape(y, (B, 32, P, P))


# speedup vs baseline: 4.1625x; 1.0092x over previous
import functools
import numpy as np
import jax
import jax.numpy as jnp
from jax import lax
from jax.experimental import pallas as pl
from jax.experimental.pallas import tpu as pltpu
from jax.experimental.pallas import tpu_sc as plsc

B = 128
NB = 202
NCH = 200
P = 25
NPIX = P * P
PADN = 640
NCHUNK = PADN // 16
CB = 40
NBLK = NCH // CB
NWORK = 32
BPW = B // NWORK

_rng = np.random.RandomState(0)
_size = int(NPIX * 0.08)
_idx_all = np.stack(
    [_rng.choice(np.arange(NPIX), replace=False, size=_size) for _ in range(B)],
    axis=0)
_forced_np = np.zeros((B, 1, PADN), dtype=np.int32)
_forced_np[np.arange(B)[:, None], 0, _idx_all] = 1
_FORCED = _forced_np

_maskx_np = np.zeros((3, PADN), dtype=np.float32)
_masky_np = np.zeros((3, PADN), dtype=np.float32)
for _d in range(3):
    for _p in range(PADN):
        _y, _x = _p // P, _p % P
        if 0 <= (_p % P) + _d - 1 < P:
            _maskx_np[_d, _p] = 1.0
        if _p < NPIX and 0 <= (_p // P) + _d - 1 < P:
            _masky_np[_d, _p] = 1.0
_MASKX = _maskx_np
_MASKY = _masky_np


def _sc_body(b0, nb_half, x_rows, forced, xo, sp_v, forced_v, m_v, rz_v,
             onepos_v, g_v, in_v0, in_v1, out_v0, out_v1, in_sem, out_sem):
    bpw = nb_half // NWORK
    wid = lax.axis_index("s") * 2 + lax.axis_index("c")
    lane = lax.iota(jnp.int32, 16)

    def batch_body(bi, _):
        bl = wid * bpw + bi
        b = b0 + bl
        pltpu.sync_copy(x_rows.at[b, pl.ds(NCH, 1)], sp_v)
        pltpu.sync_copy(forced.at[b], forced_v)
        zero16 = jnp.zeros((16,), jnp.int32)
        cvec = plsc.load_gather(sp_v, [zero16, zero16 + 312])
        c_is0 = cvec == 0.0
        last = jnp.zeros((16,), jnp.int32) + 15

        def chunk_a(j, carry):
            c1, c0 = carry
            base = j * 16
            pv = lane + base
            spj = sp_v[0, pl.ds(base, 16)]
            fj = forced_v[0, pl.ds(base, 16)]
            inb = pv < NPIX
            mbool = ((spj == cvec) | c_is0 | (fj > 0)) & inb
            mj = mbool.astype(jnp.int32)
            zj = (inb & (~mbool)).astype(jnp.int32)
            cum1 = plsc.cumsum(mj) + c1
            cum0 = plsc.cumsum(zj) + c0
            m_v[pl.ds(base, 16)] = mj
            rz_v[pl.ds(base, 16)] = cum0 - zj
            plsc.store_scatter(onepos_v, [cum1 - mj], pv, mask=mbool)
            return (jnp.take_along_axis(cum1, last, axis=0),
                    jnp.take_along_axis(cum0, last, axis=0))

        n1vec, _unused = lax.fori_loop(
            0, NCHUNK, chunk_a,
            (jnp.zeros((16,), jnp.int32), jnp.zeros((16,), jnp.int32)))

        @plsc.parallel_loop(0, NCHUNK, unroll=8)
        def chunk_b(j):
            base = j * 16
            pv = lane + base
            mj = m_v[pl.ds(base, 16)]
            t = lax.rem(rz_v[pl.ds(base, 16)], n1vec)
            src = plsc.load_gather(onepos_v, [t])
            g_v[pl.ds(base, 16)] = jnp.where(mj > 0, pv, src)

        in_bufs = (in_v0, in_v1)
        out_bufs = (out_v0, out_v1)

        def start_in(cb, buf):
            return pltpu.async_copy(
                x_rows.at[b, pl.ds(cb * CB, CB)], buf, in_sem)

        in_h = {0: start_in(0, in_bufs[0])}
        out_h = {}
        for cb in range(NBLK):
            ib = in_bufs[cb % 2]
            ob = out_bufs[cb % 2]
            in_h[cb % 2].wait()
            if cb + 1 < NBLK:
                in_h[(cb + 1) % 2] = start_in(cb + 1, in_bufs[(cb + 1) % 2])
            if cb % 2 in out_h:
                out_h[cb % 2].wait()

            @plsc.parallel_loop(0, CB * NCHUNK, unroll=8)
            def gather_t(t, ib=ib, ob=ob):
                k = t // NCHUNK
                base = (t - k * NCHUNK) * 16
                gj = g_v[pl.ds(base, 16)]
                vals = plsc.load_gather(
                    ib, [jnp.zeros((16,), jnp.int32) + k, gj])
                ob[k, pl.ds(base, 16)] = vals

            out_h[cb % 2] = pltpu.async_copy(
                ob, xo.at[bl, pl.ds(cb * CB, CB)], out_sem)
        out_h[(NBLK - 1) % 2].wait()
        out_h[(NBLK - 2) % 2].wait()
        return 0

    lax.fori_loop(0, bpw, batch_body, 0)


@functools.cache
def _sc_gather(b0, nb_half):
    mesh = plsc.VectorSubcoreMesh(core_axis_name="c", subcore_axis_name="s")
    return pl.kernel(
        functools.partial(_sc_body, b0, nb_half),
        mesh=mesh,
        compiler_params=pltpu.CompilerParams(needs_layout_passes=False),
        out_type=jax.ShapeDtypeStruct((nb_half, NCH, PADN), jnp.float32),
        scratch_types=[
            pltpu.VMEM((1, NPIX), jnp.float32),
            pltpu.VMEM((1, PADN), jnp.int32),
            pltpu.VMEM((PADN,), jnp.int32),
            pltpu.VMEM((PADN,), jnp.int32),
            pltpu.VMEM((PADN,), jnp.int32),
            pltpu.VMEM((PADN,), jnp.int32),
            pltpu.VMEM((CB, NPIX), jnp.float32),
            pltpu.VMEM((CB, NPIX), jnp.float32),
            pltpu.VMEM((CB, PADN), jnp.float32),
            pltpu.VMEM((CB, PADN), jnp.float32),
            pltpu.SemaphoreType.DMA,
            pltpu.SemaphoreType.DMA,
        ],
    )


def _rot(a, s):
    if s > 0:
        return jnp.concatenate([a[:, s:], a[:, :s]], axis=1)
    if s < 0:
        return jnp.concatenate([a[:, s:], a[:, :PADN + s]], axis=1)
    return a


CONV_BB = 8


def _conv_body(xo_ref, wt_ref, maskx_ref, masky_ref, b_ref, out_ref):
    for bb in range(CONV_BB):
        x2 = xo_ref[bb]
        us = [_rot(x2, dx) * maskx_ref[dx + 1][None, :] for dx in (-1, 0, 1)]
        acc = b_ref[...].astype(jnp.float32) * jnp.ones(
            (32, PADN), jnp.float32)
        for dy in (-1, 0, 1):
            part = jnp.zeros((32, PADN), jnp.float32)
            for dx in (-1, 0, 1):
                k = (dy + 1) * 3 + (dx + 1)
                part = part + lax.dot_general(
                    wt_ref[k], us[dx + 1], (((1,), (0,)), ((), ())),
                    preferred_element_type=jnp.float32)
            acc = acc + _rot(part, dy * P) * masky_ref[dy + 1][None, :]
        out_ref[bb] = acc[:, :NPIX]


@functools.cache
def _conv(nb):
    return pl.pallas_call(
    _conv_body,
    grid=(nb // CONV_BB,),
    in_specs=[
        pl.BlockSpec((CONV_BB, NCH, PADN), lambda i: (i, 0, 0)),
        pl.BlockSpec((9, 32, NCH), lambda i: (0, 0, 0)),
        pl.BlockSpec((3, PADN), lambda i: (0, 0)),
        pl.BlockSpec((3, PADN), lambda i: (0, 0)),
        pl.BlockSpec((32, 1), lambda i: (0, 0)),
    ],
    out_specs=pl.BlockSpec((CONV_BB, 32, NPIX), lambda i: (i, 0, 0)),
    out_shape=jax.ShapeDtypeStruct((nb, 32, NPIX), jnp.float32),
    )


def kernel(x, W, b):
    x3 = jnp.reshape(x, (B, NB, NPIX))
    wt = jnp.reshape(jnp.transpose(W, (2, 3, 0, 1)), (9, 32, NCH))
    mx, my = jnp.asarray(_MASKX), jnp.asarray(_MASKY)
    b2 = jnp.reshape(b, (32, 1))
    forced = jnp.asarray(_FORCED)
    half = B // 2
    xo_a = _sc_gather(0, half)(x3, forced)
    xo_b = _sc_gather(half, half)(x3, forced)
    y_a = _conv(half)(xo_a, wt, mx, my, b2)
    y_b = _conv(half)(xo_b, wt, mx, my, b2)
    y = jnp.concatenate([y_a, y_b], axis=0)
    return jnp.reshape(y, (B, 32, P, P))
